# Initial kernel scaffold; baseline (speedup 1.0000x reference)
import jax
import jax.numpy as jnp
from jax.experimental import pallas as pl


def kernel(x, edge_index, selected_idx, edge_label, ln1_g, ln1_b, w1_0, b1_0, w2_0, b2_0, w1_1, b1_1, w2_1, b2_1, w1_2, b1_2, w2_2, b2_2, ln2_g, ln2_b, ffn_w1, ffn_b1, ffn_w2, ffn_b2):
    raise NotImplementedError("write your pallas kernel here")


# speedup vs baseline: 1.9972x; 1.9972x over previous
import functools

import jax
import jax.numpy as jnp
from jax import lax
from jax.experimental import pallas as pl
from jax.experimental.pallas import tpu as pltpu
from jax.experimental.pallas import tpu_sc as plsc

N = 10000
E = 320000
D = 128
Bq = 100
Lq = 20

NW = 32
EPW = E // NW
NPW = 313
NPAD = NW * NPW
TRASH = NPW
TBLW = (NPW + 1) * D
GC = 128
SC_C = 128
FB = 2048
DBLK = 2000
CAP = E + FB

_mesh = plsc.VectorSubcoreMesh(core_axis_name="c", subcore_axis_name="s")


def _wid():
    return lax.axis_index("s") * 2 + lax.axis_index("c")


@functools.partial(
    pl.kernel,
    out_type=(
        jax.ShapeDtypeStruct((NW * CAP,), jnp.int32),
        jax.ShapeDtypeStruct((NW * CAP,), jnp.int32),
        jax.ShapeDtypeStruct((NW * 16,), jnp.int32),
    ),
    scratch_types=[
        pltpu.VMEM((DBLK,), jnp.int32),
        pltpu.VMEM((FB + 16,), jnp.int32),
        pltpu.VMEM((FB + 16,), jnp.int32),
        pltpu.VMEM((16,), jnp.int32),
    ],
    mesh=_mesh,
    compiler_params=pltpu.CompilerParams(needs_layout_passes=False),
)
def _partition(dst_hbm, eid_hbm, ldst_hbm, cnt_hbm, dbuf, st_eid, st_ld, cbuf):
    w = _wid()
    lo = w * NPW

    def _init(i, _):
        st_eid[pl.ds(i * 16, 16)] = jnp.zeros((16,), jnp.int32)
        st_ld[pl.ds(i * 16, 16)] = jnp.full((16,), TRASH, jnp.int32)
        return 0

    lax.fori_loop(0, (FB + 16) // 16, _init, 0)

    def _scan_block(b, carry):
        off, hoff = carry
        pltpu.sync_copy(dst_hbm.at[pl.ds(pl.multiple_of(b * DBLK, 8), DBLK)], dbuf)

        def _grp(j, c):
            off, hoff = c
            v = dbuf[pl.ds(j * 16, 16)]
            m = (v >= lo) & (v < lo + NPW)
            mi = m.astype(jnp.int32)
            eidv = (b * DBLK + j * 16) + lax.iota(jnp.int32, 16)
            pos = off + plsc.cumsum(mi) - 1
            plsc.store_scatter(st_eid, [pos], eidv, mask=m)
            plsc.store_scatter(st_ld, [pos], v - lo, mask=m)
            off2 = off + jnp.sum(mi)
            do_flush = off2 >= FB

            @pl.when(do_flush)
            def _():
                fo = pl.multiple_of(w * CAP + hoff, 8)
                pltpu.sync_copy(st_eid.at[pl.ds(0, FB)],
                                eid_hbm.at[pl.ds(fo, FB)])
                pltpu.sync_copy(st_ld.at[pl.ds(0, FB)],
                                ldst_hbm.at[pl.ds(fo, FB)])
                te = st_eid[pl.ds(FB, 16)]
                tl = st_ld[pl.ds(FB, 16)]
                st_eid[pl.ds(0, 16)] = te
                st_ld[pl.ds(0, 16)] = tl

            off3 = jnp.where(do_flush, off2 - FB, off2)
            hoff2 = jnp.where(do_flush, hoff + FB, hoff)
            return (off3, hoff2)

        return lax.fori_loop(0, DBLK // 16, _grp, (off, hoff))

    off, hoff = lax.fori_loop(0, E // DBLK, _scan_block, (0, 0))
    fo = pl.multiple_of(w * CAP + hoff, 8)
    pltpu.sync_copy(st_eid.at[pl.ds(0, FB)], eid_hbm.at[pl.ds(fo, FB)])
    pltpu.sync_copy(st_ld.at[pl.ds(0, FB)], ldst_hbm.at[pl.ds(fo, FB)])
    cbuf[pl.ds(0, 16)] = jnp.broadcast_to(hoff + FB, (16,)).astype(jnp.int32)
    pltpu.sync_copy(cbuf, cnt_hbm.at[pl.ds(pl.multiple_of(w * 16, 8), 16)])


@functools.partial(
    pl.kernel,
    out_type=(
        jax.ShapeDtypeStruct((E, D), jnp.float32),
        jax.ShapeDtypeStruct((E, D), jnp.float32),
    ),
    scratch_types=[
        pltpu.VMEM((GC,), jnp.int32),
        pltpu.VMEM((GC,), jnp.int32),
        pltpu.VMEM((GC, D), jnp.float32),
        pltpu.VMEM((GC, D), jnp.float32),
        pltpu.SemaphoreType.DMA,
        pltpu.SemaphoreType.DMA,
    ],
    mesh=_mesh,
    compiler_params=pltpu.CompilerParams(needs_layout_passes=False),
)
def _gather(p_hbm, q_hbm, dst_hbm, src_hbm, pd_hbm, qs_hbm,
            di, si, prow, qrow, semp, semq):
    w = _wid()
    base = w * EPW

    def _chunk(i, _):
        o = pl.multiple_of(base + i * GC, 8)
        pltpu.sync_copy(dst_hbm.at[pl.ds(o, GC)], di)
        pltpu.sync_copy(src_hbm.at[pl.ds(o, GC)], si)
        cp = pltpu.async_copy(p_hbm.at[di], prow, semp)
        cq = pltpu.async_copy(q_hbm.at[si], qrow, semq)
        cp.wait()
        cq.wait()
        pltpu.sync_copy(prow, pd_hbm.at[pl.ds(o, GC)])
        pltpu.sync_copy(qrow, qs_hbm.at[pl.ds(o, GC)])
        return 0

    lax.fori_loop(0, EPW // GC, _chunk, 0)
    o = pl.multiple_of(base + (EPW // GC) * GC, 8)
    pltpu.sync_copy(dst_hbm.at[pl.ds(o, 16)], di.at[pl.ds(0, 16)])
    pltpu.sync_copy(src_hbm.at[pl.ds(o, 16)], si.at[pl.ds(0, 16)])
    cp = pltpu.async_copy(p_hbm.at[di.at[pl.ds(0, 16)]], prow.at[pl.ds(0, 16)], semp)
    cq = pltpu.async_copy(q_hbm.at[si.at[pl.ds(0, 16)]], qrow.at[pl.ds(0, 16)], semq)
    cp.wait()
    cq.wait()
    pltpu.sync_copy(prow.at[pl.ds(0, 16)], pd_hbm.at[pl.ds(o, 16)])
    pltpu.sync_copy(qrow.at[pl.ds(0, 16)], qs_hbm.at[pl.ds(o, 16)])


@functools.partial(
    pl.kernel,
    out_type=jax.ShapeDtypeStruct((NPAD * D,), jnp.float32),
    scratch_types=[
        pltpu.VMEM((TBLW,), jnp.float32),
        pltpu.VMEM((SC_C,), jnp.int32),
        pltpu.VMEM((SC_C,), jnp.int32),
        pltpu.VMEM((SC_C, D), jnp.float32),
        pltpu.VMEM((NW * 16,), jnp.int32),
        pltpu.SemaphoreType.DMA,
    ],
    mesh=_mesh,
    compiler_params=pltpu.CompilerParams(needs_layout_passes=False),
)
def _scatter_max(r_hbm, eid_hbm, ldst_hbm, cnt_hbm, agg_hbm,
                 tbl, ev, lv, rows, cv, sem):
    w = _wid()

    def _zero(i, _):
        tbl[pl.ds(i * 16, 16)] = jnp.zeros((16,), jnp.float32)
        return 0

    lax.fori_loop(0, TBLW // 16, _zero, 0)
    pltpu.sync_copy(cnt_hbm, cv)
    total = cv[pl.ds(w * 16, 16)][0]

    def _chunk(i, _):
        boff = pl.multiple_of(w * CAP + i * SC_C, 8)
        pltpu.sync_copy(eid_hbm.at[pl.ds(boff, SC_C)], ev)
        pltpu.sync_copy(ldst_hbm.at[pl.ds(boff, SC_C)], lv)
        pltpu.async_copy(r_hbm.at[ev], rows, sem).wait()

        def _grp(g, _):
            lvec = lv[pl.ds(g * 16, 16)] * D
            for t in range(16):
                roff = lvec[t]
                e = g * 16 + t
                for k in range(D // 16):
                    a = tbl[pl.ds(roff + k * 16, 16)]
                    gv = rows[e, pl.ds(k * 16, 16)]
                    tbl[pl.ds(roff + k * 16, 16)] = jnp.maximum(a, gv)
            return 0

        lax.fori_loop(0, SC_C // 16, _grp, 0)
        return 0

    lax.fori_loop(0, total // SC_C, _chunk, 0)
    pltpu.sync_copy(tbl.at[pl.ds(0, NPW * D)],
                    agg_hbm.at[pl.ds(pl.multiple_of(w * NPW * D, 8), NPW * D)])


@functools.partial(
    pl.kernel,
    out_type=jax.ShapeDtypeStruct((NW * 64, D), jnp.float32),
    scratch_types=[
        pltpu.VMEM((64,), jnp.int32),
        pltpu.VMEM((64, D), jnp.float32),
        pltpu.SemaphoreType.DMA,
    ],
    mesh=_mesh,
    compiler_params=pltpu.CompilerParams(needs_layout_passes=False),
)
def _select(y_hbm, gidx_hbm, out_hbm, gi, rows, sem):
    w = _wid()
    pltpu.sync_copy(gidx_hbm.at[pl.ds(pl.multiple_of(w * 64, 8), 64)], gi)
    pltpu.async_copy(y_hbm.at[gi], rows, sem).wait()
    pltpu.sync_copy(rows, out_hbm.at[pl.ds(w * 64, 64)])


def _ln(x, g, b):
    mu = jnp.mean(x, axis=1, keepdims=True)
    var = jnp.mean((x - mu) * (x - mu), axis=1, keepdims=True)
    return (x - mu) * lax.rsqrt(var + 1e-5) * g + b


def _node_prep_body(x_ref, g_ref, b_ref, w1_ref, b1_ref, p_ref, q_ref, xn_ref,
                    agg_ref=None):
    xn = x_ref[...]
    if agg_ref is not None:
        xn = xn + agg_ref[...]
    h = _ln(xn, g_ref[...], b_ref[...])
    wb = w1_ref[D:, :]
    wa = w1_ref[:D, :] - wb
    p_ref[...] = jnp.dot(h, wa, preferred_element_type=jnp.float32) + b1_ref[...]
    q_ref[...] = jnp.dot(h, wb, preferred_element_type=jnp.float32)
    xn_ref[...] = xn


_node_out = (
    jax.ShapeDtypeStruct((N, D), jnp.float32),
    jax.ShapeDtypeStruct((N, D), jnp.float32),
    jax.ShapeDtypeStruct((N, D), jnp.float32),
)


def _node_prep0(x, g, b, w1, b1):
    return pl.pallas_call(_node_prep_body, out_shape=_node_out)(x, g, b, w1, b1)


def _node_prep(x, g, b, w1, b1, agg):
    body = functools.partial(_node_prep_body)

    def f(x_ref, g_ref, b_ref, w1_ref, b1_ref, agg_ref, p_ref, q_ref, xn_ref):
        _node_prep_body(x_ref, g_ref, b_ref, w1_ref, b1_ref, p_ref, q_ref, xn_ref,
                        agg_ref=agg_ref)

    return pl.pallas_call(f, out_shape=_node_out)(x, g, b, w1, b1, agg)


EBLK = 4000


def _edge_body(pd_ref, qs_ref, w2_ref, b2_ref, r_ref):
    z = jnp.maximum(pd_ref[...] + qs_ref[...], 0.0)
    m = jnp.dot(z, w2_ref[...], preferred_element_type=jnp.float32) + b2_ref[...]
    r_ref[...] = jnp.maximum(m, 0.0)


def _edge_mlp(pd, qs, w2, b2):
    grid = (E // EBLK,)
    return pl.pallas_call(
        _edge_body,
        grid=grid,
        in_specs=[
            pl.BlockSpec((EBLK, D), lambda i: (i, 0)),
            pl.BlockSpec((EBLK, D), lambda i: (i, 0)),
            pl.BlockSpec((D, D), lambda i: (0, 0)),
            pl.BlockSpec((1, D), lambda i: (0, 0)),
        ],
        out_specs=pl.BlockSpec((EBLK, D), lambda i: (i, 0)),
        out_shape=jax.ShapeDtypeStruct((E, D), jnp.float32),
    )(pd, qs, w2, b2)


def _ffn_body(x_ref, agg_ref, g_ref, b_ref, w1_ref, b1_ref, w2_ref, b2_ref, y_ref):
    xn = x_ref[...] + agg_ref[...]
    h = _ln(xn, g_ref[...], b_ref[...])
    h = jnp.maximum(jnp.dot(h, w1_ref[...], preferred_element_type=jnp.float32)
                    + b1_ref[...], 0.0)
    y_ref[...] = (jnp.dot(h, w2_ref[...], preferred_element_type=jnp.float32)
                  + b2_ref[...] + xn)


def _ffn(x, agg, g, b, w1, b1, w2, b2):
    return pl.pallas_call(
        _ffn_body, out_shape=jax.ShapeDtypeStruct((N, D), jnp.float32),
    )(x, agg, g, b, w1, b1, w2, b2)


def kernel(x, edge_index, selected_idx, edge_label, ln1_g, ln1_b,
           w1_0, b1_0, w2_0, b2_0, w1_1, b1_1, w2_1, b2_1,
           w1_2, b1_2, w2_2, b2_2, ln2_g, ln2_b,
           ffn_w1, ffn_b1, ffn_w2, ffn_b2):
    src = edge_index[0].astype(jnp.int32)
    dst = edge_index[1].astype(jnp.int32)
    g1 = ln1_g.reshape(1, D)
    c1 = ln1_b.reshape(1, D)
    g2 = ln2_g.reshape(1, D)
    c2 = ln2_b.reshape(1, D)

    eid, ldst, cnt = _partition(dst)

    convs = [(w1_0, b1_0, w2_0, b2_0), (w1_1, b1_1, w2_1, b2_1),
             (w1_2, b1_2, w2_2, b2_2)]
    agg = None
    for (w1, b1, w2, b2) in convs:
        if agg is None:
            p, q, xn = _node_prep0(x, g1, c1, w1, b1.reshape(1, D))
        else:
            p, q, xn = _node_prep(x, g1, c1, w1, b1.reshape(1, D), agg)
        pd, qs = _gather(p, q, dst, src)
        r = _edge_mlp(pd, qs, w2, b2.reshape(1, D))
        aggf = _scatter_max(r, eid, ldst, cnt)
        agg = aggf.reshape(NPAD, D)[:N]
        x = xn

    y = _ffn(x, agg, g2, c2, ffn_w1, ffn_b1.reshape(1, D),
             ffn_w2, ffn_b2.reshape(1, D))

    gidx = (selected_idx.astype(jnp.int32)
            + jnp.arange(Bq, dtype=jnp.int32)[:, None] * (N // Bq)).reshape(-1)
    gidx = jnp.concatenate([gidx, jnp.zeros((NW * 64 - Bq * Lq,), jnp.int32)])
    rows = _select(y, gidx)
    return rows[:Bq * Lq].reshape(Bq, Lq, D)


# speedup vs baseline: 2.5010x; 1.2523x over previous
import functools

import jax
import jax.numpy as jnp
from jax import lax
from jax.experimental import pallas as pl
from jax.experimental.pallas import tpu as pltpu
from jax.experimental.pallas import tpu_sc as plsc

N = 10000
E = 320000
D = 128
Bq = 100
Lq = 20

NW = 32
EPW = E // NW
NPW = 313
NPAD = NW * NPW
TRASH = NPW
TBLW = (NPW + 1) * D
GC = 128
SC_C = 128
FB = 2048
DBLK = 2000
CAP = E + FB

_mesh = plsc.VectorSubcoreMesh(core_axis_name="c", subcore_axis_name="s")


def _wid():
    return lax.axis_index("s") * 2 + lax.axis_index("c")


@functools.partial(
    pl.kernel,
    out_type=(
        jax.ShapeDtypeStruct((NW * CAP,), jnp.int32),
        jax.ShapeDtypeStruct((NW * CAP,), jnp.int32),
        jax.ShapeDtypeStruct((NW * 16,), jnp.int32),
    ),
    scratch_types=[
        pltpu.VMEM((DBLK,), jnp.int32),
        pltpu.VMEM((FB + 16,), jnp.int32),
        pltpu.VMEM((FB + 16,), jnp.int32),
        pltpu.VMEM((16,), jnp.int32),
    ],
    mesh=_mesh,
    compiler_params=pltpu.CompilerParams(needs_layout_passes=False),
)
def _partition(dst_hbm, eid_hbm, ldst_hbm, cnt_hbm, dbuf, st_eid, st_ld, cbuf):
    w = _wid()
    lo = w * NPW

    def _init(i, _):
        st_eid[pl.ds(i * 16, 16)] = jnp.zeros((16,), jnp.int32)
        st_ld[pl.ds(i * 16, 16)] = jnp.full((16,), TRASH, jnp.int32)
        return 0

    lax.fori_loop(0, (FB + 16) // 16, _init, 0)

    def _scan_block(b, carry):
        off, hoff = carry
        pltpu.sync_copy(dst_hbm.at[pl.ds(pl.multiple_of(b * DBLK, 8), DBLK)], dbuf)

        def _grp(j, c):
            off, hoff = c
            v = dbuf[pl.ds(j * 16, 16)]
            m = (v >= lo) & (v < lo + NPW)
            eidv = (b * DBLK + j * 16) + lax.iota(jnp.int32, 16)
            plsc.store_compressed(st_eid.at[pl.ds(off, 16)], eidv, mask=m)
            plsc.store_compressed(st_ld.at[pl.ds(off, 16)], v - lo, mask=m)
            off2 = off + plsc.all_reduce_population_count(m)[0]
            do_flush = off2 >= FB

            @pl.when(do_flush)
            def _():
                fo = pl.multiple_of(w * CAP + hoff, 8)
                pltpu.sync_copy(st_eid.at[pl.ds(0, FB)],
                                eid_hbm.at[pl.ds(fo, FB)])
                pltpu.sync_copy(st_ld.at[pl.ds(0, FB)],
                                ldst_hbm.at[pl.ds(fo, FB)])
                te = st_eid[pl.ds(FB, 16)]
                tl = st_ld[pl.ds(FB, 16)]
                st_eid[pl.ds(0, 16)] = te
                st_ld[pl.ds(0, 16)] = tl

            off3 = jnp.where(do_flush, off2 - FB, off2)
            hoff2 = jnp.where(do_flush, hoff + FB, hoff)
            return (off3, hoff2)

        return lax.fori_loop(0, DBLK // 16, _grp, (off, hoff))

    off, hoff = lax.fori_loop(0, E // DBLK, _scan_block, (0, 0))
    fo = pl.multiple_of(w * CAP + hoff, 8)
    pltpu.sync_copy(st_eid.at[pl.ds(0, FB)], eid_hbm.at[pl.ds(fo, FB)])
    pltpu.sync_copy(st_ld.at[pl.ds(0, FB)], ldst_hbm.at[pl.ds(fo, FB)])
    cbuf[pl.ds(0, 16)] = jnp.broadcast_to(hoff + FB, (16,)).astype(jnp.int32)
    pltpu.sync_copy(cbuf, cnt_hbm.at[pl.ds(pl.multiple_of(w * 16, 8), 16)])


NCH = EPW // GC


@functools.partial(
    pl.kernel,
    out_type=jax.ShapeDtypeStruct((E, D), jnp.float32),
    scratch_types=[
        pltpu.VMEM((GC,), jnp.int32),
        pltpu.VMEM((GC,), jnp.int32),
        pltpu.VMEM((GC,), jnp.int32),
        pltpu.VMEM((GC,), jnp.int32),
        pltpu.VMEM((GC, D), jnp.float32),
        pltpu.VMEM((GC, D), jnp.float32),
        pltpu.VMEM((GC, D), jnp.float32),
        pltpu.VMEM((GC, D), jnp.float32),
        pltpu.SemaphoreType.DMA,
        pltpu.SemaphoreType.DMA,
    ],
    mesh=_mesh,
    compiler_params=pltpu.CompilerParams(needs_layout_passes=False),
)
def _gather(p_hbm, q_hbm, dst_hbm, src_hbm, z_hbm,
            di0, si0, di1, si1, p0, q0, p1, q1, g0, g1):
    w = _wid()
    base = w * EPW
    bufs = ((di0, si0, p0, q0, g0), (di1, si1, p1, q1, g1))

    pltpu.sync_copy(dst_hbm.at[pl.ds(pl.multiple_of(base, 8), GC)], di0)
    pltpu.sync_copy(src_hbm.at[pl.ds(pl.multiple_of(base, 8), GC)], si0)
    pltpu.sync_copy(dst_hbm.at[pl.ds(pl.multiple_of(base + GC, 8), GC)], di1)
    pltpu.sync_copy(src_hbm.at[pl.ds(pl.multiple_of(base + GC, 8), GC)], si1)
    pltpu.async_copy(p_hbm.at[di0], p0, g0)
    pltpu.async_copy(q_hbm.at[si0], q0, g0)

    def _outer(o, _):
        for b in range(2):
            dib, sib, pb, qb, gb = bufs[b]
            dio, sio, po, qo, go = bufs[1 - b]
            c = 2 * o + b
            pltpu.make_async_copy(p_hbm.at[dib], pb, gb).wait()
            pltpu.make_async_copy(q_hbm.at[sib], qb, gb).wait()
            pltpu.async_copy(p_hbm.at[dio], po, go)
            pltpu.async_copy(q_hbm.at[sio], qo, go)
            cc = jnp.minimum(c + 2, NCH - 1)
            o2 = pl.multiple_of(base + cc * GC, 8)
            pltpu.sync_copy(dst_hbm.at[pl.ds(o2, GC)], dib)
            pltpu.sync_copy(src_hbm.at[pl.ds(o2, GC)], sib)

            def _add(r, _):
                for k in range(D // 16):
                    pb[r, pl.ds(k * 16, 16)] = (pb[r, pl.ds(k * 16, 16)]
                                                + qb[r, pl.ds(k * 16, 16)])
                return 0

            lax.fori_loop(0, GC, _add, 0)
            pltpu.sync_copy(pb, z_hbm.at[pl.ds(pl.multiple_of(base + c * GC, 8), GC)])
        return 0

    lax.fori_loop(0, NCH // 2, _outer, 0)
    pltpu.make_async_copy(p_hbm.at[di0], p0, g0).wait()
    pltpu.make_async_copy(q_hbm.at[si0], q0, g0).wait()
    o = pl.multiple_of(base + NCH * GC, 8)
    pltpu.sync_copy(dst_hbm.at[pl.ds(o, 16)], di0.at[pl.ds(0, 16)])
    pltpu.sync_copy(src_hbm.at[pl.ds(o, 16)], si0.at[pl.ds(0, 16)])
    cp = pltpu.async_copy(p_hbm.at[di0.at[pl.ds(0, 16)]], p0.at[pl.ds(0, 16)], g0)
    cq = pltpu.async_copy(q_hbm.at[si0.at[pl.ds(0, 16)]], q0.at[pl.ds(0, 16)], g1)
    cp.wait()
    cq.wait()

    def _addt(r, _):
        for k in range(D // 16):
            p0[r, pl.ds(k * 16, 16)] = (p0[r, pl.ds(k * 16, 16)]
                                        + q0[r, pl.ds(k * 16, 16)])
        return 0

    lax.fori_loop(0, 16, _addt, 0)
    pltpu.sync_copy(p0.at[pl.ds(0, 16)], z_hbm.at[pl.ds(o, 16)])


@functools.partial(
    pl.kernel,
    out_type=jax.ShapeDtypeStruct((NPAD * D,), jnp.float32),
    scratch_types=[
        pltpu.VMEM((TBLW,), jnp.float32),
        pltpu.VMEM((SC_C,), jnp.int32),
        pltpu.VMEM((SC_C,), jnp.int32),
        pltpu.VMEM((SC_C,), jnp.int32),
        pltpu.VMEM((SC_C,), jnp.int32),
        pltpu.VMEM((SC_C, D), jnp.float32),
        pltpu.VMEM((SC_C, D), jnp.float32),
        pltpu.VMEM((NW * 16,), jnp.int32),
        pltpu.SemaphoreType.DMA,
        pltpu.SemaphoreType.DMA,
    ],
    mesh=_mesh,
    compiler_params=pltpu.CompilerParams(needs_layout_passes=False),
)
def _scatter_max(r_hbm, eid_hbm, ldst_hbm, cnt_hbm, agg_hbm,
                 tbl, ev0, lv0, ev1, lv1, rows0, rows1, cv, s0, s1):
    w = _wid()

    def _zero(i, _):
        tbl[pl.ds(i * 16, 16)] = jnp.zeros((16,), jnp.float32)
        return 0

    lax.fori_loop(0, TBLW // 16, _zero, 0)
    pltpu.sync_copy(cnt_hbm, cv)
    total = cv[pl.ds(w * 16, 16)][0]
    nch = total // SC_C
    cbase = w * CAP
    bufs = ((ev0, lv0, rows0, s0), (ev1, lv1, rows1, s1))

    pltpu.sync_copy(eid_hbm.at[pl.ds(pl.multiple_of(cbase, 8), SC_C)], ev0)
    pltpu.sync_copy(ldst_hbm.at[pl.ds(pl.multiple_of(cbase, 8), SC_C)], lv0)
    pltpu.sync_copy(eid_hbm.at[pl.ds(pl.multiple_of(cbase + SC_C, 8), SC_C)], ev1)
    pltpu.sync_copy(ldst_hbm.at[pl.ds(pl.multiple_of(cbase + SC_C, 8), SC_C)], lv1)
    pltpu.async_copy(r_hbm.at[ev0], rows0, s0)

    def _outer(o, _):
        for b in range(2):
            evb, lvb, rowsb, sb = bufs[b]
            evo, lvo, rowso, so = bufs[1 - b]
            c = 2 * o + b
            pltpu.make_async_copy(r_hbm.at[evb], rowsb, sb).wait()
            pltpu.async_copy(r_hbm.at[evo], rowso, so)

            def _grp(g, _):
                lvec = lvb[pl.ds(g * 16, 16)] * D
                for t in range(16):
                    roff = lvec[t]
                    e = g * 16 + t
                    for k in range(D // 16):
                        a = tbl[pl.ds(roff + k * 16, 16)]
                        gv = rowsb[e, pl.ds(k * 16, 16)]
                        tbl[pl.ds(roff + k * 16, 16)] = jnp.maximum(a, gv)
                return 0

            lax.fori_loop(0, SC_C // 16, _grp, 0)
            cc = jnp.minimum(c + 2, nch - 1)
            o2 = pl.multiple_of(cbase + cc * SC_C, 8)
            pltpu.sync_copy(eid_hbm.at[pl.ds(o2, SC_C)], evb)
            pltpu.sync_copy(ldst_hbm.at[pl.ds(o2, SC_C)], lvb)
        return 0

    lax.fori_loop(0, nch // 2, _outer, 0)
    pltpu.make_async_copy(r_hbm.at[ev0], rows0, s0).wait()
    pltpu.sync_copy(tbl.at[pl.ds(0, NPW * D)],
                    agg_hbm.at[pl.ds(pl.multiple_of(w * NPW * D, 8), NPW * D)])


@functools.partial(
    pl.kernel,
    out_type=jax.ShapeDtypeStruct((NW * 64, D), jnp.float32),
    scratch_types=[
        pltpu.VMEM((64,), jnp.int32),
        pltpu.VMEM((64, D), jnp.float32),
        pltpu.SemaphoreType.DMA,
    ],
    mesh=_mesh,
    compiler_params=pltpu.CompilerParams(needs_layout_passes=False),
)
def _select(y_hbm, gidx_hbm, out_hbm, gi, rows, sem):
    w = _wid()
    pltpu.sync_copy(gidx_hbm.at[pl.ds(pl.multiple_of(w * 64, 8), 64)], gi)
    pltpu.async_copy(y_hbm.at[gi], rows, sem).wait()
    pltpu.sync_copy(rows, out_hbm.at[pl.ds(w * 64, 64)])


def _ln(x, g, b):
    mu = jnp.mean(x, axis=1, keepdims=True)
    var = jnp.mean((x - mu) * (x - mu), axis=1, keepdims=True)
    return (x - mu) * lax.rsqrt(var + 1e-5) * g + b


def _node_prep_body(x_ref, g_ref, b_ref, w1_ref, b1_ref, p_ref, q_ref, xn_ref,
                    agg_ref=None):
    xn = x_ref[...]
    if agg_ref is not None:
        xn = xn + agg_ref[...]
    h = _ln(xn, g_ref[...], b_ref[...])
    wb = w1_ref[D:, :]
    wa = w1_ref[:D, :] - wb
    p_ref[...] = jnp.dot(h, wa, preferred_element_type=jnp.float32) + b1_ref[...]
    q_ref[...] = jnp.dot(h, wb, preferred_element_type=jnp.float32)
    xn_ref[...] = xn


_node_out = (
    jax.ShapeDtypeStruct((N, D), jnp.float32),
    jax.ShapeDtypeStruct((N, D), jnp.float32),
    jax.ShapeDtypeStruct((N, D), jnp.float32),
)


def _node_prep0(x, g, b, w1, b1):
    return pl.pallas_call(_node_prep_body, out_shape=_node_out)(x, g, b, w1, b1)


def _node_prep(x, g, b, w1, b1, agg):
    body = functools.partial(_node_prep_body)

    def f(x_ref, g_ref, b_ref, w1_ref, b1_ref, agg_ref, p_ref, q_ref, xn_ref):
        _node_prep_body(x_ref, g_ref, b_ref, w1_ref, b1_ref, p_ref, q_ref, xn_ref,
                        agg_ref=agg_ref)

    return pl.pallas_call(f, out_shape=_node_out)(x, g, b, w1, b1, agg)


EBLK = 4000


def _edge_body(z_ref, w2_ref, b2_ref, r_ref):
    z = jnp.maximum(z_ref[...], 0.0)
    m = jnp.dot(z, w2_ref[...], preferred_element_type=jnp.float32) + b2_ref[...]
    r_ref[...] = jnp.maximum(m, 0.0)


def _edge_mlp(z, w2, b2):
    grid = (E // EBLK,)
    return pl.pallas_call(
        _edge_body,
        grid=grid,
        in_specs=[
            pl.BlockSpec((EBLK, D), lambda i: (i, 0)),
            pl.BlockSpec((D, D), lambda i: (0, 0)),
            pl.BlockSpec((1, D), lambda i: (0, 0)),
        ],
        out_specs=pl.BlockSpec((EBLK, D), lambda i: (i, 0)),
        out_shape=jax.ShapeDtypeStruct((E, D), jnp.float32),
    )(z, w2, b2)


def _ffn_body(x_ref, agg_ref, g_ref, b_ref, w1_ref, b1_ref, w2_ref, b2_ref, y_ref):
    xn = x_ref[...] + agg_ref[...]
    h = _ln(xn, g_ref[...], b_ref[...])
    h = jnp.maximum(jnp.dot(h, w1_ref[...], preferred_element_type=jnp.float32)
                    + b1_ref[...], 0.0)
    y_ref[...] = (jnp.dot(h, w2_ref[...], preferred_element_type=jnp.float32)
                  + b2_ref[...] + xn)


def _ffn(x, agg, g, b, w1, b1, w2, b2):
    return pl.pallas_call(
        _ffn_body, out_shape=jax.ShapeDtypeStruct((N, D), jnp.float32),
    )(x, agg, g, b, w1, b1, w2, b2)


def kernel(x, edge_index, selected_idx, edge_label, ln1_g, ln1_b,
           w1_0, b1_0, w2_0, b2_0, w1_1, b1_1, w2_1, b2_1,
           w1_2, b1_2, w2_2, b2_2, ln2_g, ln2_b,
           ffn_w1, ffn_b1, ffn_w2, ffn_b2):
    src = edge_index[0].astype(jnp.int32)
    dst = edge_index[1].astype(jnp.int32)
    g1 = ln1_g.reshape(1, D)
    c1 = ln1_b.reshape(1, D)
    g2 = ln2_g.reshape(1, D)
    c2 = ln2_b.reshape(1, D)

    eid, ldst, cnt = _partition(dst)

    convs = [(w1_0, b1_0, w2_0, b2_0), (w1_1, b1_1, w2_1, b2_1),
             (w1_2, b1_2, w2_2, b2_2)]
    agg = None
    for (w1, b1, w2, b2) in convs:
        if agg is None:
            p, q, xn = _node_prep0(x, g1, c1, w1, b1.reshape(1, D))
        else:
            p, q, xn = _node_prep(x, g1, c1, w1, b1.reshape(1, D), agg)
        z = _gather(p, q, dst, src)
        r = _edge_mlp(z, w2, b2.reshape(1, D))
        aggf = _scatter_max(r, eid, ldst, cnt)
        agg = aggf.reshape(NPAD, D)[:N]
        x = xn

    y = _ffn(x, agg, g2, c2, ffn_w1, ffn_b1.reshape(1, D),
             ffn_w2, ffn_b2.reshape(1, D))

    gidx = (selected_idx.astype(jnp.int32)
            + jnp.arange(Bq, dtype=jnp.int32)[:, None] * (N // Bq)).reshape(-1)
    gidx = jnp.concatenate([gidx, jnp.zeros((NW * 64 - Bq * Lq,), jnp.int32)])
    rows = _select(y, gidx)
    return rows[:Bq * Lq].reshape(Bq, Lq, D)


# speedup vs baseline: 2.5986x; 1.0390x over previous
import functools

import jax
import jax.numpy as jnp
from jax import lax
from jax.experimental import pallas as pl
from jax.experimental.pallas import tpu as pltpu
from jax.experimental.pallas import tpu_sc as plsc

N = 10000
E = 320000
D = 128
Bq = 100
Lq = 20

NW = 32
EPW = E // NW
NPW = 313
NPAD = NW * NPW
TRASH = NPW
TBLW = (NPW + 1) * D
GC = 128
SC_C = 128
FB = 2048
DBLK = 2000
CAP = E + FB

_mesh = plsc.VectorSubcoreMesh(core_axis_name="c", subcore_axis_name="s")


def _wid():
    return lax.axis_index("s") * 2 + lax.axis_index("c")


@functools.partial(
    pl.kernel,
    out_type=(
        jax.ShapeDtypeStruct((NW * CAP,), jnp.int32),
        jax.ShapeDtypeStruct((NW * CAP,), jnp.int32),
        jax.ShapeDtypeStruct((NW * 16,), jnp.int32),
    ),
    scratch_types=[
        pltpu.VMEM((DBLK,), jnp.int32),
        pltpu.VMEM((FB + DBLK + 16,), jnp.int32),
        pltpu.VMEM((FB + DBLK + 16,), jnp.int32),
        pltpu.VMEM((16,), jnp.int32),
    ],
    mesh=_mesh,
    compiler_params=pltpu.CompilerParams(needs_layout_passes=False),
)
def _partition(dst_hbm, eid_hbm, ldst_hbm, cnt_hbm, dbuf, st_eid, st_ld, cbuf):
    w = _wid()
    lo = w * NPW
    STW = (FB + DBLK + 16) // 16

    def _arm(trash_eid):
        def _init(i, _):
            st_eid[pl.ds(i * 16, 16)] = jnp.full((16,), trash_eid, jnp.int32)
            st_ld[pl.ds(i * 16, 16)] = jnp.full((16,), TRASH, jnp.int32)
            return 0
        lax.fori_loop(0, STW, _init, 0)

    _arm(0)

    def _scan_block(b, carry):
        off, hoff, mid = carry
        pltpu.sync_copy(dst_hbm.at[pl.ds(pl.multiple_of(b * DBLK, 8), DBLK)], dbuf)

        def _grp(j, off):
            v = dbuf[pl.ds(j * 16, 16)]
            m = (v >= lo) & (v < lo + NPW)
            eidv = (b * DBLK + j * 16) + lax.iota(jnp.int32, 16)
            plsc.store_compressed(st_eid.at[pl.ds(off, 16)], eidv, mask=m)
            plsc.store_compressed(st_ld.at[pl.ds(off, 16)], v - lo, mask=m)
            return off + plsc.all_reduce_population_count(m)[0]

        off = lax.fori_loop(0, DBLK // 16, _grp, off)
        do_flush = off >= FB

        @pl.when(do_flush)
        def _():
            fo = pl.multiple_of(w * CAP + hoff, 8)
            pltpu.sync_copy(st_eid.at[pl.ds(0, FB)], eid_hbm.at[pl.ds(fo, FB)])
            pltpu.sync_copy(st_ld.at[pl.ds(0, FB)], ldst_hbm.at[pl.ds(fo, FB)])

            def _mv(i, _):
                te = st_eid[pl.ds(FB + i * 16, 16)]
                tl = st_ld[pl.ds(FB + i * 16, 16)]
                st_eid[pl.ds(i * 16, 16)] = te
                st_ld[pl.ds(i * 16, 16)] = tl
                return 0

            lax.fori_loop(0, (off - FB + 31) // 16, _mv, 0)

        off = jnp.where(do_flush, off - FB, off)
        hoff = jnp.where(do_flush, hoff + FB, hoff)
        at_mid = b == (E // DBLK) // 2 - 1

        @pl.when(at_mid)
        def _():
            fo = pl.multiple_of(w * CAP + hoff, 8)
            pltpu.sync_copy(st_eid.at[pl.ds(0, FB)], eid_hbm.at[pl.ds(fo, FB)])
            pltpu.sync_copy(st_ld.at[pl.ds(0, FB)], ldst_hbm.at[pl.ds(fo, FB)])
            _arm(E - 1)

        hoff = jnp.where(at_mid, hoff + FB, hoff)
        off = jnp.where(at_mid, 0, off)
        mid = jnp.where(at_mid, hoff, mid)
        return (off, hoff, mid)

    off, hoff, mid = lax.fori_loop(0, E // DBLK, _scan_block, (0, 0, 0))
    fo = pl.multiple_of(w * CAP + hoff, 8)
    pltpu.sync_copy(st_eid.at[pl.ds(0, FB)], eid_hbm.at[pl.ds(fo, FB)])
    pltpu.sync_copy(st_ld.at[pl.ds(0, FB)], ldst_hbm.at[pl.ds(fo, FB)])
    iot = lax.iota(jnp.int32, 16)
    cbuf[pl.ds(0, 16)] = jnp.where(iot == 1,
                                   jnp.broadcast_to(mid, (16,)),
                                   jnp.broadcast_to(hoff + FB, (16,))).astype(jnp.int32)
    pltpu.sync_copy(cbuf, cnt_hbm.at[pl.ds(pl.multiple_of(w * 16, 8), 16)])


E2 = E // 2
EPW2 = E2 // NW
NC2 = EPW2 // GC
GTAIL = EPW2 - NC2 * GC


def _make_gather(sbase):
    @functools.partial(
        pl.kernel,
        out_type=jax.ShapeDtypeStruct((E2, D), jnp.float32),
        scratch_types=[
            pltpu.VMEM((GC,), jnp.int32),
            pltpu.VMEM((GC,), jnp.int32),
            pltpu.VMEM((GC,), jnp.int32),
            pltpu.VMEM((GC,), jnp.int32),
            pltpu.VMEM((GC, D), jnp.float32),
            pltpu.VMEM((GC, D), jnp.float32),
            pltpu.VMEM((GC, D), jnp.float32),
            pltpu.VMEM((GC, D), jnp.float32),
            pltpu.SemaphoreType.DMA,
            pltpu.SemaphoreType.DMA,
        ],
        mesh=_mesh,
        compiler_params=pltpu.CompilerParams(needs_layout_passes=False),
    )
    def _gather(p_hbm, q_hbm, dst_hbm, src_hbm, z_hbm,
                di0, si0, di1, si1, p0, q0, p1, q1, g0, g1):
        w = _wid()
        base = sbase + w * EPW2
        zb = w * EPW2
        bufs = ((di0, si0, p0, q0, g0), (di1, si1, p1, q1, g1))

        def _add_rows(pb, qb, n):
            def _add(r, _):
                for k in range(D // 16):
                    pb[r, pl.ds(k * 16, 16)] = (pb[r, pl.ds(k * 16, 16)]
                                                + qb[r, pl.ds(k * 16, 16)])
                return 0
            lax.fori_loop(0, n, _add, 0)

        pltpu.sync_copy(dst_hbm.at[pl.ds(pl.multiple_of(base, 8), GC)], di0)
        pltpu.sync_copy(src_hbm.at[pl.ds(pl.multiple_of(base, 8), GC)], si0)
        pltpu.sync_copy(dst_hbm.at[pl.ds(pl.multiple_of(base + GC, 8), GC)], di1)
        pltpu.sync_copy(src_hbm.at[pl.ds(pl.multiple_of(base + GC, 8), GC)], si1)
        pltpu.async_copy(p_hbm.at[di0], p0, g0)
        pltpu.async_copy(q_hbm.at[si0], q0, g0)

        def _outer(o, _):
            for b in range(2):
                dib, sib, pb, qb, gb = bufs[b]
                dio, sio, po, qo, go = bufs[1 - b]
                c = 2 * o + b
                pltpu.make_async_copy(p_hbm.at[dib], pb, gb).wait()
                pltpu.make_async_copy(q_hbm.at[sib], qb, gb).wait()
                pltpu.async_copy(p_hbm.at[dio], po, go)
                pltpu.async_copy(q_hbm.at[sio], qo, go)
                cc = jnp.minimum(c + 2, NC2 - 1)
                o2 = pl.multiple_of(base + cc * GC, 8)
                pltpu.sync_copy(dst_hbm.at[pl.ds(o2, GC)], dib)
                pltpu.sync_copy(src_hbm.at[pl.ds(o2, GC)], sib)
                _add_rows(pb, qb, GC)
                pltpu.sync_copy(pb, z_hbm.at[pl.ds(pl.multiple_of(zb + c * GC, 8), GC)])
            return 0

        lax.fori_loop(0, NC2 // 2, _outer, 0)
        pltpu.make_async_copy(p_hbm.at[di0], p0, g0).wait()
        pltpu.make_async_copy(q_hbm.at[si0], q0, g0).wait()
        _add_rows(p0, q0, GC)
        pltpu.sync_copy(p0, z_hbm.at[pl.ds(pl.multiple_of(zb + (NC2 - 1) * GC, 8), GC)])
        o = pl.multiple_of(base + NC2 * GC, 8)
        zo = pl.multiple_of(zb + NC2 * GC, 8)
        pltpu.sync_copy(dst_hbm.at[pl.ds(o, GTAIL)], di1.at[pl.ds(0, GTAIL)])
        pltpu.sync_copy(src_hbm.at[pl.ds(o, GTAIL)], si1.at[pl.ds(0, GTAIL)])
        cp = pltpu.async_copy(p_hbm.at[di1.at[pl.ds(0, GTAIL)]],
                              p1.at[pl.ds(0, GTAIL)], g0)
        cq = pltpu.async_copy(q_hbm.at[si1.at[pl.ds(0, GTAIL)]],
                              q1.at[pl.ds(0, GTAIL)], g1)
        cp.wait()
        cq.wait()
        _add_rows(p1, q1, GTAIL)
        pltpu.sync_copy(p1.at[pl.ds(0, GTAIL)], z_hbm.at[pl.ds(zo, GTAIL)])

    return _gather


_gather_s0 = _make_gather(0)
_gather_s1 = _make_gather(E2)


def _make_scatter(phase):
    scratch = [
        pltpu.VMEM((TBLW,), jnp.float32),
        pltpu.VMEM((SC_C,), jnp.int32),
        pltpu.VMEM((SC_C,), jnp.int32),
        pltpu.VMEM((SC_C,), jnp.int32),
        pltpu.VMEM((SC_C,), jnp.int32),
        pltpu.VMEM((SC_C, D), jnp.float32),
        pltpu.VMEM((SC_C, D), jnp.float32),
        pltpu.VMEM((NW * 16,), jnp.int32),
        pltpu.SemaphoreType.DMA,
        pltpu.SemaphoreType.DMA,
    ]

    def _body(r_hbm, eid_hbm, ldst_hbm, cnt_hbm, agg_in, agg_hbm,
              tbl, ev0, lv0, ev1, lv1, rows0, rows1, cv, s0, s1):
        w = _wid()
        wrow = pl.multiple_of(w * NPW * D, 8)
        if phase == 0:
            def _zero(i, _):
                tbl[pl.ds(i * 16, 16)] = jnp.zeros((16,), jnp.float32)
                return 0
            lax.fori_loop(0, TBLW // 16, _zero, 0)
        else:
            pltpu.sync_copy(agg_in.at[pl.ds(wrow, NPW * D)], tbl.at[pl.ds(0, NPW * D)])

            def _zt(i, _):
                tbl[pl.ds(NPW * D + i * 16, 16)] = jnp.zeros((16,), jnp.float32)
                return 0
            lax.fori_loop(0, D // 16, _zt, 0)
        pltpu.sync_copy(cnt_hbm, cv)
        cw = cv[pl.ds(w * 16, 16)]
        if phase == 0:
            start, count, eoff = 0, cw[1], 0
        else:
            start, count, eoff = cw[1], cw[0] - cw[1], E2
        nch = count // SC_C
        cbase = w * CAP + start
        bufs = ((ev0, lv0, rows0, s0), (ev1, lv1, rows1, s1))

        def _load_idx(c_off, evb, lvb):
            o2 = pl.multiple_of(cbase + c_off, 8)
            pltpu.sync_copy(eid_hbm.at[pl.ds(o2, SC_C)], evb)
            pltpu.sync_copy(ldst_hbm.at[pl.ds(o2, SC_C)], lvb)
            if phase == 1:
                for j in range(SC_C // 16):
                    evb[pl.ds(j * 16, 16)] = evb[pl.ds(j * 16, 16)] - eoff

        _load_idx(0, ev0, lv0)
        _load_idx(SC_C, ev1, lv1)
        pltpu.async_copy(r_hbm.at[ev0], rows0, s0)

        def _outer(o, _):
            for b in range(2):
                evb, lvb, rowsb, sb = bufs[b]
                evo, lvo, rowso, so = bufs[1 - b]
                c = 2 * o + b
                pltpu.make_async_copy(r_hbm.at[evb], rowsb, sb).wait()
                pltpu.async_copy(r_hbm.at[evo], rowso, so)

                def _grp(g, _):
                    lvec = lvb[pl.ds(g * 16, 16)] * D
                    for t in range(16):
                        roff = lvec[t]
                        e = g * 16 + t
                        for k in range(D // 16):
                            a = tbl[pl.ds(roff + k * 16, 16)]
                            gv = rowsb[e, pl.ds(k * 16, 16)]
                            tbl[pl.ds(roff + k * 16, 16)] = jnp.maximum(a, gv)
                    return 0

                lax.fori_loop(0, SC_C // 16, _grp, 0)
                cc = jnp.minimum(c + 2, nch - 1)
                _load_idx(cc * SC_C, evb, lvb)
            return 0

        lax.fori_loop(0, nch // 2, _outer, 0)
        pltpu.make_async_copy(r_hbm.at[ev0], rows0, s0).wait()
        pltpu.sync_copy(tbl.at[pl.ds(0, NPW * D)], agg_hbm.at[pl.ds(wrow, NPW * D)])

    if phase == 0:
        def _body0(r_hbm, eid_hbm, ldst_hbm, cnt_hbm, agg_hbm, *scr):
            _body(r_hbm, eid_hbm, ldst_hbm, cnt_hbm, None, agg_hbm, *scr)
        body = _body0
    else:
        body = _body
    return functools.partial(
        pl.kernel,
        out_type=jax.ShapeDtypeStruct((NPAD * D,), jnp.float32),
        scratch_types=scratch,
        mesh=_mesh,
        compiler_params=pltpu.CompilerParams(needs_layout_passes=False),
    )(body)


_scatter_p0 = _make_scatter(0)
_scatter_p1 = _make_scatter(1)


@functools.partial(
    pl.kernel,
    out_type=jax.ShapeDtypeStruct((NW * 64, D), jnp.float32),
    scratch_types=[
        pltpu.VMEM((64,), jnp.int32),
        pltpu.VMEM((64, D), jnp.float32),
        pltpu.SemaphoreType.DMA,
    ],
    mesh=_mesh,
    compiler_params=pltpu.CompilerParams(needs_layout_passes=False),
)
def _select(y_hbm, gidx_hbm, out_hbm, gi, rows, sem):
    w = _wid()
    pltpu.sync_copy(gidx_hbm.at[pl.ds(pl.multiple_of(w * 64, 8), 64)], gi)
    pltpu.async_copy(y_hbm.at[gi], rows, sem).wait()
    pltpu.sync_copy(rows, out_hbm.at[pl.ds(w * 64, 64)])


def _ln(x, g, b):
    mu = jnp.mean(x, axis=1, keepdims=True)
    var = jnp.mean((x - mu) * (x - mu), axis=1, keepdims=True)
    return (x - mu) * lax.rsqrt(var + 1e-5) * g + b


def _node_prep_body(x_ref, g_ref, b_ref, w1_ref, b1_ref, p_ref, q_ref, xn_ref,
                    agg_ref=None):
    xn = x_ref[...]
    if agg_ref is not None:
        xn = xn + agg_ref[...]
    h = _ln(xn, g_ref[...], b_ref[...])
    wb = w1_ref[D:, :]
    wa = w1_ref[:D, :] - wb
    p_ref[...] = jnp.dot(h, wa, preferred_element_type=jnp.float32) + b1_ref[...]
    q_ref[...] = jnp.dot(h, wb, preferred_element_type=jnp.float32)
    xn_ref[...] = xn


_node_out = (
    jax.ShapeDtypeStruct((N, D), jnp.float32),
    jax.ShapeDtypeStruct((N, D), jnp.float32),
    jax.ShapeDtypeStruct((N, D), jnp.float32),
)


def _node_prep0(x, g, b, w1, b1):
    return pl.pallas_call(_node_prep_body, out_shape=_node_out)(x, g, b, w1, b1)


def _node_prep(x, g, b, w1, b1, agg):
    body = functools.partial(_node_prep_body)

    def f(x_ref, g_ref, b_ref, w1_ref, b1_ref, agg_ref, p_ref, q_ref, xn_ref):
        _node_prep_body(x_ref, g_ref, b_ref, w1_ref, b1_ref, p_ref, q_ref, xn_ref,
                        agg_ref=agg_ref)

    return pl.pallas_call(f, out_shape=_node_out)(x, g, b, w1, b1, agg)


EBLK = 4000


def _edge_body(z_ref, w2_ref, b2_ref, r_ref):
    z = jnp.maximum(z_ref[...], 0.0)
    m = jnp.dot(z, w2_ref[...], preferred_element_type=jnp.float32) + b2_ref[...]
    r_ref[...] = jnp.maximum(m, 0.0)


def _edge_mlp(z, w2, b2):
    grid = (E2 // EBLK,)
    return pl.pallas_call(
        _edge_body,
        grid=grid,
        in_specs=[
            pl.BlockSpec((EBLK, D), lambda i: (i, 0)),
            pl.BlockSpec((D, D), lambda i: (0, 0)),
            pl.BlockSpec((1, D), lambda i: (0, 0)),
        ],
        out_specs=pl.BlockSpec((EBLK, D), lambda i: (i, 0)),
        out_shape=jax.ShapeDtypeStruct((E2, D), jnp.float32),
    )(z, w2, b2)


def _ffn_body(x_ref, agg_ref, g_ref, b_ref, w1_ref, b1_ref, w2_ref, b2_ref, y_ref):
    xn = x_ref[...] + agg_ref[...]
    h = _ln(xn, g_ref[...], b_ref[...])
    h = jnp.maximum(jnp.dot(h, w1_ref[...], preferred_element_type=jnp.float32)
                    + b1_ref[...], 0.0)
    y_ref[...] = (jnp.dot(h, w2_ref[...], preferred_element_type=jnp.float32)
                  + b2_ref[...] + xn)


def _ffn(x, agg, g, b, w1, b1, w2, b2):
    return pl.pallas_call(
        _ffn_body, out_shape=jax.ShapeDtypeStruct((N, D), jnp.float32),
    )(x, agg, g, b, w1, b1, w2, b2)


def kernel(x, edge_index, selected_idx, edge_label, ln1_g, ln1_b,
           w1_0, b1_0, w2_0, b2_0, w1_1, b1_1, w2_1, b2_1,
           w1_2, b1_2, w2_2, b2_2, ln2_g, ln2_b,
           ffn_w1, ffn_b1, ffn_w2, ffn_b2):
    src = edge_index[0].astype(jnp.int32)
    dst = edge_index[1].astype(jnp.int32)
    g1 = ln1_g.reshape(1, D)
    c1 = ln1_b.reshape(1, D)
    g2 = ln2_g.reshape(1, D)
    c2 = ln2_b.reshape(1, D)

    eid, ldst, cnt = _partition(dst)

    convs = [(w1_0, b1_0, w2_0, b2_0), (w1_1, b1_1, w2_1, b2_1),
             (w1_2, b1_2, w2_2, b2_2)]
    agg = None
    for (w1, b1, w2, b2) in convs:
        if agg is None:
            p, q, xn = _node_prep0(x, g1, c1, w1, b1.reshape(1, D))
        else:
            p, q, xn = _node_prep(x, g1, c1, w1, b1.reshape(1, D), agg)
        z1 = _gather_s0(p, q, dst, src)
        z2 = _gather_s1(p, q, dst, src)
        b2r = b2.reshape(1, D)
        r1 = _edge_mlp(z1, w2, b2r)
        r2 = _edge_mlp(z2, w2, b2r)
        a1 = _scatter_p0(r1, eid, ldst, cnt)
        aggf = _scatter_p1(r2, eid, ldst, cnt, a1)
        agg = aggf.reshape(NPAD, D)[:N]
        x = xn

    y = _ffn(x, agg, g2, c2, ffn_w1, ffn_b1.reshape(1, D),
             ffn_w2, ffn_b2.reshape(1, D))

    gidx = (selected_idx.astype(jnp.int32)
            + jnp.arange(Bq, dtype=jnp.int32)[:, None] * (N // Bq)).reshape(-1)
    gidx = jnp.concatenate([gidx, jnp.zeros((NW * 64 - Bq * Lq,), jnp.int32)])
    rows = _select(y, gidx)
    return rows[:Bq * Lq].reshape(Bq, Lq, D)
